# baseline (speedup 1.0000x reference)
import functools

import jax
import jax.numpy as jnp
from jax import lax
from jax.experimental import pallas as pl
from jax.experimental.pallas import tpu as pltpu
from jax.experimental.pallas import tpu_sc as plsc

_LANES = 16


def _gmpool_sc(n_rows, w, n_out):
    info = plsc.get_sparse_core_info()
    nc, ns = info.num_cores, info.num_subcores
    nw = nc * ns
    rows_per_w = n_rows // nw
    r = 224
    n_chunks = rows_per_w // r
    assert rows_per_w % r == 0 and n_rows % nw == 0

    mesh = plsc.VectorSubcoreMesh(core_axis_name="c", subcore_axis_name="s")

    @functools.partial(
        pl.kernel,
        mesh=mesh,
        compiler_params=pltpu.CompilerParams(needs_layout_passes=False),
        out_type=jax.ShapeDtypeStruct((n_rows * n_out,), jnp.float32),
        scratch_types=[
            pltpu.VMEM((r, w), jnp.float32),
            pltpu.VMEM((r * n_out,), jnp.float32),
            pltpu.VMEM((n_out, _LANES), jnp.int32),
            pltpu.SemaphoreType.DMA,
            pltpu.SemaphoreType.DMA,
        ],
    )
    def k(x_hbm, idx_hbm, out_hbm, xbuf, obuf, idxbuf, insem, outsem):
        cid = lax.axis_index("c")
        sid = lax.axis_index("s")
        wid = sid * nc + cid
        row0 = wid * rows_per_w

        pltpu.sync_copy(idx_hbm, idxbuf)
        idv = [idxbuf[kk, :] for kk in range(_LANES)]

        def chunk_body(g, carry):
            base_row = row0 + g * r
            pltpu.sync_copy(x_hbm.at[pl.ds(base_row, r)], xbuf)

            def row_body(rr, c2):
                row_idx = jnp.full((_LANES,), rr, jnp.int32)
                acc = plsc.load_gather(xbuf, [row_idx, idv[0]])
                for kk in range(1, _LANES):
                    acc = jnp.maximum(
                        acc, plsc.load_gather(xbuf, [row_idx, idv[kk]])
                    )
                obuf[pl.ds(rr * n_out, n_out)] = acc
                return c2

            lax.fori_loop(0, r, row_body, 0, unroll=2)

            pltpu.sync_copy(
                obuf, out_hbm.at[pl.ds(base_row * n_out, r * n_out)]
            )
            return carry

        lax.fori_loop(0, n_chunks, chunk_body, 0)

    return k


def kernel(x, indices):
    b, c, h, w = x.shape
    n_out = indices.shape[1]
    n_rows = b * c * h
    k = _gmpool_sc(n_rows, w, n_out)
    out = k(x.reshape(n_rows, w), indices)
    return out.reshape(b, c, h, n_out)


# speedup vs baseline: 1.2300x; 1.2300x over previous
import functools

import jax
import jax.numpy as jnp
from jax import lax
from jax.experimental import pallas as pl
from jax.experimental.pallas import tpu as pltpu
from jax.experimental.pallas import tpu_sc as plsc

_LANES = 16


def _gmpool_sc(n_rows, w, n_out):
    info = plsc.get_sparse_core_info()
    nc, ns = info.num_cores, info.num_subcores
    nw = nc * ns
    rows_per_w = n_rows // nw
    r = 224
    n_chunks = rows_per_w // r
    assert rows_per_w % r == 0 and n_rows % nw == 0 and n_chunks % 2 == 0

    mesh = plsc.VectorSubcoreMesh(core_axis_name="c", subcore_axis_name="s")

    @functools.partial(
        pl.kernel,
        mesh=mesh,
        compiler_params=pltpu.CompilerParams(needs_layout_passes=False),
        out_type=jax.ShapeDtypeStruct((n_rows * n_out,), jnp.float32),
        scratch_types=[
            pltpu.VMEM((2, r, w), jnp.float32),
            pltpu.VMEM((2, r * n_out), jnp.float32),
            pltpu.VMEM((n_out, _LANES), jnp.int32),
            pltpu.SemaphoreType.DMA,
            pltpu.SemaphoreType.DMA,
            pltpu.SemaphoreType.DMA,
            pltpu.SemaphoreType.DMA,
        ],
    )
    def k(x_hbm, idx_hbm, out_hbm, xbuf, obuf, idxbuf, is0, is1, os0, os1):
        cid = lax.axis_index("c")
        sid = lax.axis_index("s")
        wid = sid * nc + cid
        row0 = wid * rows_per_w
        isems = (is0, is1)
        osems = (os0, os1)

        pltpu.sync_copy(idx_hbm, idxbuf)
        idv = [idxbuf[kk, :] for kk in range(_LANES)]

        def in_copy(g, b):
            return pltpu.make_async_copy(
                x_hbm.at[pl.ds((row0 + g * r), r)], xbuf.at[b], isems[b]
            )

        def out_copy(g, b):
            return pltpu.make_async_copy(
                obuf.at[b],
                out_hbm.at[pl.ds((row0 + g * r) * n_out, r * n_out)],
                osems[b],
            )

        def compute(b):
            def row_body(rr, c2):
                row_idx = jnp.full((_LANES,), rr, jnp.int32)
                vals = [
                    plsc.load_gather(xbuf.at[b], [row_idx, idv[kk]])
                    for kk in range(_LANES)
                ]
                while len(vals) > 1:
                    vals = [
                        jnp.maximum(vals[2 * i], vals[2 * i + 1])
                        for i in range(len(vals) // 2)
                    ]
                obuf[b, pl.ds(rr * n_out, n_out)] = vals[0]
                return c2

            lax.fori_loop(0, r, row_body, 0, unroll=4)

        in_copy(0, 0).start()

        def pair_body(p, carry):
            for b in range(2):
                g = 2 * p + b
                in_copy(g, b).wait()

                @pl.when(g + 1 < n_chunks)
                def _():
                    in_copy(g + 1, 1 - b).start()

                @pl.when(g >= 2)
                def _():
                    out_copy(g - 2, b).wait()

                compute(b)
                out_copy(g, b).start()
            return carry

        lax.fori_loop(0, n_chunks // 2, pair_body, 0)
        out_copy(n_chunks - 2, 0).wait()
        out_copy(n_chunks - 1, 1).wait()

    return k


def kernel(x, indices):
    b, c, h, w = x.shape
    n_out = indices.shape[1]
    n_rows = b * c * h
    k = _gmpool_sc(n_rows, w, n_out)
    out = k(x.reshape(n_rows, w), indices)
    return out.reshape(b, c, h, n_out)


# speedup vs baseline: 1.8574x; 1.5101x over previous
import functools

import jax
import jax.numpy as jnp
from jax import lax
from jax.experimental import pallas as pl
from jax.experimental.pallas import tpu as pltpu
from jax.experimental.pallas import tpu_sc as plsc

_LANES = 16


def _gmpool_sc(n_slabs, h, w, n_out):
    info = plsc.get_sparse_core_info()
    nc, ns = info.num_cores, info.num_subcores
    nw = nc * ns
    slabs_per_w = n_slabs // nw
    n_pairs = slabs_per_w // 2
    oslab = h * n_out
    assert n_slabs % nw == 0 and slabs_per_w % 4 == 0
    assert (2 * oslab) % 128 == 0

    mesh = plsc.VectorSubcoreMesh(core_axis_name="c", subcore_axis_name="s")

    @functools.partial(
        pl.kernel,
        mesh=mesh,
        compiler_params=pltpu.CompilerParams(needs_layout_passes=False),
        out_type=jax.ShapeDtypeStruct((n_slabs * oslab,), jnp.float32),
        scratch_types=[
            pltpu.VMEM((2, h, w), jnp.float32),
            pltpu.VMEM((2, 2 * oslab), jnp.float32),
            pltpu.VMEM((n_out, _LANES), jnp.int32),
            pltpu.SemaphoreType.DMA,
            pltpu.SemaphoreType.DMA,
            pltpu.SemaphoreType.DMA,
            pltpu.SemaphoreType.DMA,
        ],
    )
    def k(x_hbm, idx_hbm, out_hbm, xbuf, obuf, idxbuf, is0, is1, os0, os1):
        cid = lax.axis_index("c")
        sid = lax.axis_index("s")
        wid = sid * nc + cid
        slab0 = wid * slabs_per_w
        isems = (is0, is1)
        osems = (os0, os1)

        pltpu.sync_copy(idx_hbm, idxbuf)
        idv = [idxbuf[kk, :] for kk in range(_LANES)]

        def in_copy(g, b):
            return pltpu.make_async_copy(
                x_hbm.at[slab0 + g], xbuf.at[b], isems[b]
            )

        def out_copy(p, pb):
            return pltpu.make_async_copy(
                obuf.at[pb],
                out_hbm.at[pl.ds((slab0 + 2 * p) * oslab, 2 * oslab)],
                osems[pb],
            )

        def compute(b, pb):
            def row_body(rr, c2):
                row_idx = jnp.full((_LANES,), rr, jnp.int32)
                vals = [
                    plsc.load_gather(xbuf.at[b], [row_idx, idv[kk]])
                    for kk in range(_LANES)
                ]
                while len(vals) > 1:
                    vals = [
                        jnp.maximum(vals[2 * i], vals[2 * i + 1])
                        for i in range(len(vals) // 2)
                    ]
                obuf[pb, pl.ds(b * oslab + rr * n_out, n_out)] = vals[0]
                return c2

            lax.fori_loop(0, h, row_body, 0, unroll=4)

        in_copy(0, 0).start()

        def quad_body(q, carry):
            for pb in range(2):
                p = 2 * q + pb

                @pl.when(p >= 2)
                def _():
                    out_copy(p - 2, pb).wait()

                for b in range(2):
                    g = 2 * p + b
                    in_copy(g, b).wait()

                    @pl.when(g + 1 < slabs_per_w)
                    def _():
                        in_copy(g + 1, 1 - b).start()

                    compute(b, pb)
                out_copy(p, pb).start()
            return carry

        lax.fori_loop(0, n_pairs // 2, quad_body, 0)
        out_copy(n_pairs - 2, 0).wait()
        out_copy(n_pairs - 1, 1).wait()

    return k


def kernel(x, indices):
    b, c, h, w = x.shape
    n_out = indices.shape[1]
    n_slabs = b * c
    k = _gmpool_sc(n_slabs, h, w, n_out)
    out = k(x.reshape(n_slabs, h, w), indices)
    return out.reshape(b, c, h, n_out)


# speedup vs baseline: 4.5865x; 2.4693x over previous
import functools

import jax
import jax.numpy as jnp
from jax import lax
from jax.experimental import pallas as pl
from jax.experimental.pallas import tpu as pltpu
from jax.experimental.pallas import tpu_sc as plsc

_LANES = 16


def _gmpool_sc(n_slabs, c, w, n_out):
    info = plsc.get_sparse_core_info()
    nc, ns = info.num_cores, info.num_subcores
    nw = nc * ns
    slabs_per_w = n_slabs // nw
    oslab = c * n_out
    assert n_slabs % nw == 0 and slabs_per_w % 2 == 0
    assert oslab % 128 == 0

    mesh = plsc.VectorSubcoreMesh(core_axis_name="c", subcore_axis_name="s")

    @functools.partial(
        pl.kernel,
        mesh=mesh,
        compiler_params=pltpu.CompilerParams(needs_layout_passes=False),
        out_type=jax.ShapeDtypeStruct((n_slabs, n_out, c), jnp.float32),
        scratch_types=[
            pltpu.VMEM((2, c, w), jnp.float32),
            pltpu.VMEM((n_out, c), jnp.float32),
            pltpu.VMEM((n_out, c), jnp.float32),
            pltpu.VMEM((n_out, _LANES), jnp.int32),
            pltpu.SemaphoreType.DMA,
            pltpu.SemaphoreType.DMA,
            pltpu.SemaphoreType.DMA,
            pltpu.SemaphoreType.DMA,
        ],
    )
    def k(x_hbm, idx_hbm, out_hbm, xbuf, ob0, ob1, idxbuf, is0, is1, os0, os1):
        cid = lax.axis_index("c")
        sid = lax.axis_index("s")
        wid = sid * nc + cid
        slab0 = wid * slabs_per_w
        isems = (is0, is1)
        osems = (os0, os1)
        obufs = (ob0, ob1)

        pltpu.sync_copy(idx_hbm, idxbuf)
        idv = [idxbuf[kk, :] for kk in range(_LANES)]
        lane = lax.iota(jnp.int32, _LANES)

        def in_copy(g, b):
            return pltpu.make_async_copy(
                x_hbm.at[slab0 + g], xbuf.at[b], isems[b]
            )

        def out_copy(g, b):
            return pltpu.make_async_copy(
                obufs[b], out_hbm.at[slab0 + g], osems[b]
            )

        def compute(b):
            def row_body(rr, c2):
                row_idx = jnp.full((_LANES,), rr, jnp.int32)
                vals = [
                    plsc.load_gather(xbuf.at[b], [row_idx, idv[kk]])
                    for kk in range(_LANES)
                ]
                while len(vals) > 1:
                    vals = [
                        jnp.maximum(vals[2 * i], vals[2 * i + 1])
                        for i in range(len(vals) // 2)
                    ]
                plsc.store_scatter(obufs[b], [lane, row_idx], vals[0])
                return c2

            lax.fori_loop(0, c, row_body, 0, unroll=4)

        in_copy(0, 0).start()

        def pair_body(p, carry):
            for b in range(2):
                g = 2 * p + b
                in_copy(g, b).wait()

                @pl.when(g + 1 < slabs_per_w)
                def _():
                    in_copy(g + 1, 1 - b).start()

                @pl.when(g >= 2)
                def _():
                    out_copy(g - 2, b).wait()

                compute(b)
                out_copy(g, b).start()
            return carry

        lax.fori_loop(0, slabs_per_w // 2, pair_body, 0)
        out_copy(slabs_per_w - 2, 0).wait()
        out_copy(slabs_per_w - 1, 1).wait()

    return k


def kernel(x, indices):
    b, c, h, w = x.shape
    n_out = indices.shape[1]
    n_slabs = b * h
    k = _gmpool_sc(n_slabs, c, w, n_out)
    xt = x.transpose(0, 2, 1, 3).reshape(n_slabs, c, w)
    out = k(xt, indices)
    return out.reshape(b, h, n_out, c).transpose(0, 3, 1, 2)


# speedup vs baseline: 4.6169x; 1.0066x over previous
import functools

import jax
import jax.numpy as jnp
from jax import lax
from jax.experimental import pallas as pl
from jax.experimental.pallas import tpu as pltpu
from jax.experimental.pallas import tpu_sc as plsc

_LANES = 16


def _gmpool_sc(n_slabs, c, w, n_out):
    info = plsc.get_sparse_core_info()
    nc, ns = info.num_cores, info.num_subcores
    nw = nc * ns
    slabs_per_w = n_slabs // nw
    oslab = c * n_out
    assert n_slabs % nw == 0 and slabs_per_w % 2 == 0
    assert oslab % 128 == 0

    mesh = plsc.VectorSubcoreMesh(core_axis_name="c", subcore_axis_name="s")

    @functools.partial(
        pl.kernel,
        mesh=mesh,
        compiler_params=pltpu.CompilerParams(needs_layout_passes=False),
        out_type=jax.ShapeDtypeStruct((n_slabs, n_out, c), jnp.float32),
        scratch_types=[
            pltpu.VMEM((2, c, w), jnp.float32),
            pltpu.VMEM((n_out, c), jnp.float32),
            pltpu.VMEM((n_out, c), jnp.float32),
            pltpu.VMEM((n_out, _LANES), jnp.int32),
            pltpu.SemaphoreType.DMA,
            pltpu.SemaphoreType.DMA,
            pltpu.SemaphoreType.DMA,
            pltpu.SemaphoreType.DMA,
            pltpu.SemaphoreType.DMA,
            pltpu.SemaphoreType.DMA,
        ],
    )
    def k(
        x_hbm, idx_hbm, out_hbm, xbuf, ob0, ob1, idxbuf,
        is0a, is0b, is1a, is1b, os0, os1,
    ):
        cid = lax.axis_index("c")
        sid = lax.axis_index("s")
        wid = sid * nc + cid
        slab0 = wid * slabs_per_w
        isems = ((is0a, is0b), (is1a, is1b))
        osems = (os0, os1)
        obufs = (ob0, ob1)
        ch = c // 2

        pltpu.sync_copy(idx_hbm, idxbuf)
        idv = [idxbuf[kk, :] for kk in range(_LANES)]
        lane = lax.iota(jnp.int32, _LANES)

        def in_half(g, b, half):
            return pltpu.make_async_copy(
                x_hbm.at[slab0 + g, pl.ds(half * ch, ch)],
                xbuf.at[b, pl.ds(half * ch, ch)],
                isems[b][half],
            )

        def in_start(g, b):
            in_half(g, b, 0).start()
            in_half(g, b, 1).start()

        def in_wait(g, b):
            in_half(g, b, 0).wait()
            in_half(g, b, 1).wait()

        def out_copy(g, b):
            return pltpu.make_async_copy(
                obufs[b], out_hbm.at[slab0 + g], osems[b]
            )

        def compute(b):
            def row_body(rr, c2):
                row_idx = jnp.full((_LANES,), rr, jnp.int32)

                def tmax(lo, hi):
                    if hi - lo == 1:
                        return plsc.load_gather(
                            xbuf.at[b], [row_idx, idv[lo]]
                        )
                    mid = (lo + hi) // 2
                    return jnp.maximum(tmax(lo, mid), tmax(mid, hi))

                plsc.store_scatter(
                    obufs[b], [lane, row_idx], tmax(0, _LANES)
                )
                return c2

            lax.fori_loop(0, c, row_body, 0, unroll=4)

        in_start(0, 0)

        def pair_body(p, carry):
            for b in range(2):
                g = 2 * p + b
                in_wait(g, b)

                @pl.when(g + 1 < slabs_per_w)
                def _():
                    in_start(g + 1, 1 - b)

                @pl.when(g >= 2)
                def _():
                    out_copy(g - 2, b).wait()

                compute(b)
                out_copy(g, b).start()
            return carry

        lax.fori_loop(0, slabs_per_w // 2, pair_body, 0)
        out_copy(slabs_per_w - 2, 0).wait()
        out_copy(slabs_per_w - 1, 1).wait()

    return k


def kernel(x, indices):
    b, c, h, w = x.shape
    n_out = indices.shape[1]
    n_slabs = b * h
    k = _gmpool_sc(n_slabs, c, w, n_out)
    xt = x.transpose(0, 2, 1, 3).reshape(n_slabs, c, w)
    out = k(xt, indices)
    return out.reshape(b, h, n_out, c).transpose(0, 3, 1, 2)
